# baseline (speedup 1.0000x reference)
import jax
import jax.numpy as jnp
from jax import lax
from jax.experimental import pallas as pl
from jax.experimental.pallas import tpu as pltpu
from jax.experimental.pallas import tpu_sc as plsc

N = 10000
E = 320000
D_IN = 128
H = 64
D_OUT = 128

NC = 2
NS = 16
NW = NC * NS
CH = 128
NCHUNK = 80
EPW = NCHUNK * CH
E_PAD = NW * EPW - E
NACC = N + 8
NPS = 624
NTAIL = N - NS * NPS

_mesh = plsc.VectorSubcoreMesh(core_axis_name="c", subcore_axis_name="s")


def _build_agg(with_deg):
    GK = 2 if with_deg else 4
    NG = NCHUNK // GK
    out_type = [jax.ShapeDtypeStruct((NC, N, H), jnp.float32)]
    scratch = [
        pltpu.VMEM((NCHUNK, CH), jnp.int32),
        pltpu.VMEM((NCHUNK, CH), jnp.int32),
        pltpu.VMEM((2, GK, CH, H), jnp.float32),
        pltpu.VMEM_SHARED((NACC, H), jnp.float32),
        pltpu.SemaphoreType.DMA,
        pltpu.SemaphoreType.DMA,
        pltpu.SemaphoreType.DMA,
        pltpu.SemaphoreType.DMA,
    ]
    if with_deg:
        out_type.append(jax.ShapeDtypeStruct((NC, N, 16), jnp.float32))
        scratch += [
            pltpu.VMEM((CH, 16), jnp.float32),
            pltpu.VMEM_SHARED((NACC, 16), jnp.float32),
        ]

    def body(*refs):
        if with_deg:
            (h_hbm, src_hbm, dst_hbm, z64_hbm, ones_hbm, z16_hbm,
             part_hbm, degp_hbm,
             src_v, dst_v, rows_v, acc_sh, sem0, sem1, ssem0, ssem1,
             ones_v, dacc_sh) = refs
        else:
            (h_hbm, src_hbm, dst_hbm, z64_hbm,
             part_hbm,
             src_v, dst_v, rows_v, acc_sh, sem0, sem1, ssem0, ssem1) = refs

        c = lax.axis_index("c")
        s = lax.axis_index("s")
        wid = c * NS + s
        r0 = pl.multiple_of(s * NPS, 8)

        pltpu.sync_copy(z64_hbm.at[pl.ds(r0, NPS)], acc_sh.at[pl.ds(r0, NPS)])
        if with_deg:
            pltpu.sync_copy(z16_hbm.at[pl.ds(r0, NPS)],
                            dacc_sh.at[pl.ds(r0, NPS)])
            pltpu.sync_copy(ones_hbm, ones_v)

        @pl.when(s == NS - 1)
        def _():
            t0 = NS * NPS
            pltpu.sync_copy(z64_hbm.at[pl.ds(t0, NTAIL)],
                            acc_sh.at[pl.ds(t0, NTAIL)])
            if with_deg:
                pltpu.sync_copy(z16_hbm.at[pl.ds(t0, NTAIL)],
                                dacc_sh.at[pl.ds(t0, NTAIL)])
        pltpu.sync_copy(src_hbm.at[wid], src_v)
        pltpu.sync_copy(dst_hbm.at[wid], dst_v)
        plsc.subcore_barrier()

        def fire_g(g, buf, sem):
            for b in range(GK):
                pltpu.async_copy(
                    h_hbm.at[src_v.at[g * GK + b]], rows_v.at[buf, b], sem)

        def wait_g(g, buf, sem):
            for b in range(GK):
                pltpu.make_async_copy(
                    h_hbm.at[src_v.at[g * GK + b]], rows_v.at[buf, b], sem
                ).wait()

        def fire_s(g, buf, sem):
            for b in range(GK):
                t = g * GK + b
                pltpu.async_copy(rows_v.at[buf, b], acc_sh.at[dst_v.at[t]],
                                 sem, add=True)
                if with_deg:
                    pltpu.async_copy(ones_v, dacc_sh.at[dst_v.at[t]], sem,
                                     add=True)

        def wait_s(g, buf, sem):
            for b in range(GK):
                t = g * GK + b
                pltpu.make_async_copy(rows_v.at[buf, b],
                                      acc_sh.at[dst_v.at[t]], sem).wait()
                if with_deg:
                    pltpu.make_async_copy(ones_v, dacc_sh.at[dst_v.at[t]],
                                          sem).wait()

        fire_g(0, 0, sem0)

        @pl.loop(0, NG - 1, step=2)
        def _(i):
            @pl.when(i > 0)
            def _():
                wait_s(i - 1, 1, ssem1)
            fire_g(i + 1, 1, sem1)
            wait_g(i, 0, sem0)
            fire_s(i, 0, ssem0)
            wait_s(i, 0, ssem0)

            @pl.when(i + 2 < NG)
            def _():
                fire_g(i + 2, 0, sem0)
            wait_g(i + 1, 1, sem1)
            fire_s(i + 1, 1, ssem1)

        wait_s(NG - 1, 1, ssem1)
        plsc.subcore_barrier()

        pltpu.sync_copy(acc_sh.at[pl.ds(r0, NPS)],
                        part_hbm.at[c, pl.ds(r0, NPS)])
        if with_deg:
            pltpu.sync_copy(dacc_sh.at[pl.ds(r0, NPS)],
                            degp_hbm.at[c, pl.ds(r0, NPS)])

        @pl.when(s == NS - 1)
        def _():
            t0 = NS * NPS
            pltpu.sync_copy(acc_sh.at[pl.ds(t0, NTAIL)],
                            part_hbm.at[c, pl.ds(t0, NTAIL)])
            if with_deg:
                pltpu.sync_copy(dacc_sh.at[pl.ds(t0, NTAIL)],
                                degp_hbm.at[c, pl.ds(t0, NTAIL)])

    return pl.kernel(body, out_type=out_type, mesh=_mesh,
                     scratch_types=scratch,
                     compiler_params=pltpu.CompilerParams(
                         use_tc_tiling_on_sc=False))


_agg_deg = _build_agg(True)
_agg = _build_agg(False)

_BN = 2000


def _enc_body(x_ref, w_ref, b_ref, o_ref):
    o_ref[...] = jnp.dot(x_ref[...], w_ref[...],
                         preferred_element_type=jnp.float32) + b_ref[...]


def _encoder(x, w, b):
    return pl.pallas_call(
        _enc_body,
        grid=(N // _BN,),
        in_specs=[pl.BlockSpec((_BN, D_IN), lambda i: (i, 0)),
                  pl.BlockSpec((D_IN, H), lambda i: (0, 0)),
                  pl.BlockSpec((1, H), lambda i: (0, 0))],
        out_specs=pl.BlockSpec((_BN, H), lambda i: (i, 0)),
        out_shape=jax.ShapeDtypeStruct((N, H), jnp.float32),
    )(x, w, b.reshape(1, H))


def _sage_core(h_ref, np_ref, dp_ref, ws_ref, wn_ref, bl_ref, g_ref, be_ref):
    deg = dp_ref[0, :, 0:1] + dp_ref[1, :, 0:1]
    inv = 1.0 / jnp.maximum(deg, 1.0)
    neigh = (np_ref[0] + np_ref[1]) * inv
    h = h_ref[...]
    z = (jnp.dot(h, ws_ref[...], preferred_element_type=jnp.float32)
         + jnp.dot(neigh, wn_ref[...], preferred_element_type=jnp.float32)
         + bl_ref[...])
    z = jnp.maximum(z, 0.0) + h
    mu = jnp.mean(z, axis=1, keepdims=True)
    var = jnp.mean((z - mu) ** 2, axis=1, keepdims=True)
    return (z - mu) / jnp.sqrt(var + 1e-5) * g_ref[...] + be_ref[...]


def _layer_body(h_ref, np_ref, dp_ref, ws_ref, wn_ref, bl_ref, g_ref, be_ref,
                o_ref):
    o_ref[...] = _sage_core(h_ref, np_ref, dp_ref, ws_ref, wn_ref, bl_ref,
                            g_ref, be_ref)


def _layer_skip_body(h_ref, np_ref, dp_ref, ws_ref, wn_ref, bl_ref, g_ref,
                     be_ref, fh_ref, wsk_ref, bsk_ref, o_ref):
    y = _sage_core(h_ref, np_ref, dp_ref, ws_ref, wn_ref, bl_ref, g_ref,
                   be_ref)
    o_ref[...] = y + jnp.dot(fh_ref[...], wsk_ref[...],
                             preferred_element_type=jnp.float32) + bsk_ref[...]


def _layer_out_body(h_ref, np_ref, dp_ref, ws_ref, wn_ref, bl_ref, g_ref,
                    be_ref, wo_ref, bo_ref, o_ref):
    y = _sage_core(h_ref, np_ref, dp_ref, ws_ref, wn_ref, bl_ref, g_ref,
                   be_ref)
    o_ref[...] = jnp.dot(y, wo_ref[...],
                         preferred_element_type=jnp.float32) + bo_ref[...]


def _base_specs():
    return [pl.BlockSpec((_BN, H), lambda i: (i, 0)),
            pl.BlockSpec((2, _BN, H), lambda i: (0, i, 0)),
            pl.BlockSpec((2, _BN, 16), lambda i: (0, i, 0)),
            pl.BlockSpec((H, H), lambda i: (0, 0)),
            pl.BlockSpec((H, H), lambda i: (0, 0)),
            pl.BlockSpec((1, H), lambda i: (0, 0)),
            pl.BlockSpec((1, H), lambda i: (0, 0)),
            pl.BlockSpec((1, H), lambda i: (0, 0))]


def _layer(h, part, degp, ws, wn, bl, g, be):
    return pl.pallas_call(
        _layer_body,
        grid=(N // _BN,),
        in_specs=_base_specs(),
        out_specs=pl.BlockSpec((_BN, H), lambda i: (i, 0)),
        out_shape=jax.ShapeDtypeStruct((N, H), jnp.float32),
    )(h, part, degp, ws, wn, bl.reshape(1, H), g.reshape(1, H),
      be.reshape(1, H))


def _layer_skip(h, part, degp, ws, wn, bl, g, be, fh, wsk, bsk):
    specs = _base_specs() + [pl.BlockSpec((_BN, H), lambda i: (i, 0)),
                             pl.BlockSpec((H, H), lambda i: (0, 0)),
                             pl.BlockSpec((1, H), lambda i: (0, 0))]
    return pl.pallas_call(
        _layer_skip_body,
        grid=(N // _BN,),
        in_specs=specs,
        out_specs=pl.BlockSpec((_BN, H), lambda i: (i, 0)),
        out_shape=jax.ShapeDtypeStruct((N, H), jnp.float32),
    )(h, part, degp, ws, wn, bl.reshape(1, H), g.reshape(1, H),
      be.reshape(1, H), fh, wsk, bsk.reshape(1, H))


def _layer_out(h, part, degp, ws, wn, bl, g, be, wo, bo):
    specs = _base_specs() + [pl.BlockSpec((H, D_OUT), lambda i: (0, 0)),
                             pl.BlockSpec((1, D_OUT), lambda i: (0, 0))]
    return pl.pallas_call(
        _layer_out_body,
        grid=(N // _BN,),
        in_specs=specs,
        out_specs=pl.BlockSpec((_BN, D_OUT), lambda i: (i, 0)),
        out_shape=jax.ShapeDtypeStruct((N, D_OUT), jnp.float32),
    )(h, part, degp, ws, wn, bl.reshape(1, H), g.reshape(1, H),
      be.reshape(1, H), wo, bo.reshape(1, D_OUT))


def kernel(x, edge_index, W_enc, b_enc, W_self_0, W_neigh_0, b_l_0, g_0, be_0,
           W_self_1, W_neigh_1, b_l_1, g_1, be_1, W_self_2, W_neigh_2, b_l_2,
           g_2, be_2, W_skip, b_skip, W_out, b_out):
    pad_src = jnp.zeros((E_PAD,), jnp.int32)
    pad_dst = jnp.full((E_PAD,), N, jnp.int32)
    src = jnp.concatenate([edge_index[0], pad_src]).reshape(NW, NCHUNK, CH)
    dst = jnp.concatenate([edge_index[1], pad_dst]).reshape(NW, NCHUNK, CH)
    z64 = jnp.zeros((N, H), jnp.float32)
    z16 = jnp.zeros((N, 16), jnp.float32)
    ones = jnp.ones((CH, 16), jnp.float32)

    h0 = _encoder(x, W_enc, b_enc)
    part0, degp = _agg_deg(h0, src, dst, z64, ones, z16)
    h1 = _layer(h0, part0, degp, W_self_0, W_neigh_0, b_l_0, g_0, be_0)
    (part1,) = _agg(h1, src, dst, z64)
    h2 = _layer_skip(h1, part1, degp, W_self_1, W_neigh_1, b_l_1, g_1, be_1,
                     h0, W_skip, b_skip)
    (part2,) = _agg(h2, src, dst, z64)
    out = _layer_out(h2, part2, degp, W_self_2, W_neigh_2, b_l_2, g_2, be_2,
                     W_out, b_out)
    return out


# speedup vs baseline: 1.0004x; 1.0004x over previous
import jax
import jax.numpy as jnp
from jax import lax
from jax.experimental import pallas as pl
from jax.experimental.pallas import tpu as pltpu
from jax.experimental.pallas import tpu_sc as plsc

N = 10000
E = 320000
D_IN = 128
H = 64
D_OUT = 128

NC = 2
NS = 16
NW = NC * NS
CH = 128
NCHUNK = 80
EPW = NCHUNK * CH
E_PAD = NW * EPW - E
NTRASH = 512
NACC = N + NTRASH
NPS = 624
NTAIL = N - NS * NPS

_mesh = plsc.VectorSubcoreMesh(core_axis_name="c", subcore_axis_name="s")


def _build_agg(with_deg):
    GK = 2 if with_deg else 4
    NG = NCHUNK // GK
    out_type = [jax.ShapeDtypeStruct((NC, N, H), jnp.float32)]
    scratch = [
        pltpu.VMEM((NCHUNK, CH), jnp.int32),
        pltpu.VMEM((NCHUNK, CH), jnp.int32),
        pltpu.VMEM((2, GK, CH, H), jnp.float32),
        pltpu.VMEM_SHARED((NACC, H), jnp.float32),
        pltpu.SemaphoreType.DMA,
        pltpu.SemaphoreType.DMA,
        pltpu.SemaphoreType.DMA,
        pltpu.SemaphoreType.DMA,
    ]
    if with_deg:
        out_type.append(jax.ShapeDtypeStruct((NC, N, 16), jnp.float32))
        scratch += [
            pltpu.VMEM((CH, 16), jnp.float32),
            pltpu.VMEM_SHARED((NACC, 16), jnp.float32),
        ]

    def body(*refs):
        if with_deg:
            (h_hbm, src_hbm, dst_hbm, z64_hbm, ones_hbm, z16_hbm,
             part_hbm, degp_hbm,
             src_v, dst_v, rows_v, acc_sh, sem0, sem1, ssem0, ssem1,
             ones_v, dacc_sh) = refs
        else:
            (h_hbm, src_hbm, dst_hbm, z64_hbm,
             part_hbm,
             src_v, dst_v, rows_v, acc_sh, sem0, sem1, ssem0, ssem1) = refs

        c = lax.axis_index("c")
        s = lax.axis_index("s")
        wid = c * NS + s
        r0 = pl.multiple_of(s * NPS, 8)

        pltpu.sync_copy(z64_hbm.at[pl.ds(r0, NPS)], acc_sh.at[pl.ds(r0, NPS)])
        if with_deg:
            pltpu.sync_copy(z16_hbm.at[pl.ds(r0, NPS)],
                            dacc_sh.at[pl.ds(r0, NPS)])
            pltpu.sync_copy(ones_hbm, ones_v)

        @pl.when(s == NS - 1)
        def _():
            t0 = NS * NPS
            pltpu.sync_copy(z64_hbm.at[pl.ds(t0, NTAIL)],
                            acc_sh.at[pl.ds(t0, NTAIL)])
            if with_deg:
                pltpu.sync_copy(z16_hbm.at[pl.ds(t0, NTAIL)],
                                dacc_sh.at[pl.ds(t0, NTAIL)])
        pltpu.sync_copy(src_hbm.at[wid], src_v)
        pltpu.sync_copy(dst_hbm.at[wid], dst_v)
        plsc.subcore_barrier()

        def fire_g(g, buf, sem):
            for b in range(GK):
                pltpu.async_copy(
                    h_hbm.at[src_v.at[g * GK + b]], rows_v.at[buf, b], sem)

        def wait_g(g, buf, sem):
            for b in range(GK):
                pltpu.make_async_copy(
                    h_hbm.at[src_v.at[g * GK + b]], rows_v.at[buf, b], sem
                ).wait()

        def fire_s(g, buf, sem):
            for b in range(GK):
                t = g * GK + b
                pltpu.async_copy(rows_v.at[buf, b], acc_sh.at[dst_v.at[t]],
                                 sem, add=True)
                if with_deg:
                    pltpu.async_copy(ones_v, dacc_sh.at[dst_v.at[t]], sem,
                                     add=True)

        def wait_s(g, buf, sem):
            for b in range(GK):
                t = g * GK + b
                pltpu.make_async_copy(rows_v.at[buf, b],
                                      acc_sh.at[dst_v.at[t]], sem).wait()
                if with_deg:
                    pltpu.make_async_copy(ones_v, dacc_sh.at[dst_v.at[t]],
                                          sem).wait()

        fire_g(0, 0, sem0)

        @pl.loop(0, NG - 1, step=2)
        def _(i):
            @pl.when(i > 0)
            def _():
                wait_s(i - 1, 1, ssem1)
            fire_g(i + 1, 1, sem1)
            wait_g(i, 0, sem0)
            fire_s(i, 0, ssem0)
            wait_s(i, 0, ssem0)

            @pl.when(i + 2 < NG)
            def _():
                fire_g(i + 2, 0, sem0)
            wait_g(i + 1, 1, sem1)
            fire_s(i + 1, 1, ssem1)

        wait_s(NG - 1, 1, ssem1)
        plsc.subcore_barrier()

        pltpu.sync_copy(acc_sh.at[pl.ds(r0, NPS)],
                        part_hbm.at[c, pl.ds(r0, NPS)])
        if with_deg:
            pltpu.sync_copy(dacc_sh.at[pl.ds(r0, NPS)],
                            degp_hbm.at[c, pl.ds(r0, NPS)])

        @pl.when(s == NS - 1)
        def _():
            t0 = NS * NPS
            pltpu.sync_copy(acc_sh.at[pl.ds(t0, NTAIL)],
                            part_hbm.at[c, pl.ds(t0, NTAIL)])
            if with_deg:
                pltpu.sync_copy(dacc_sh.at[pl.ds(t0, NTAIL)],
                                degp_hbm.at[c, pl.ds(t0, NTAIL)])

    return pl.kernel(body, out_type=out_type, mesh=_mesh,
                     scratch_types=scratch,
                     compiler_params=pltpu.CompilerParams(
                         use_tc_tiling_on_sc=False))


_agg_deg = _build_agg(True)
_agg = _build_agg(False)

_BN = 2000


def _enc_body(x_ref, w_ref, b_ref, o_ref):
    o_ref[...] = jnp.dot(x_ref[...], w_ref[...],
                         preferred_element_type=jnp.float32) + b_ref[...]


def _encoder(x, w, b):
    return pl.pallas_call(
        _enc_body,
        grid=(N // _BN,),
        in_specs=[pl.BlockSpec((_BN, D_IN), lambda i: (i, 0)),
                  pl.BlockSpec((D_IN, H), lambda i: (0, 0)),
                  pl.BlockSpec((1, H), lambda i: (0, 0))],
        out_specs=pl.BlockSpec((_BN, H), lambda i: (i, 0)),
        out_shape=jax.ShapeDtypeStruct((N, H), jnp.float32),
    )(x, w, b.reshape(1, H))


def _sage_core(h_ref, np_ref, dp_ref, ws_ref, wn_ref, bl_ref, g_ref, be_ref):
    deg = dp_ref[0, :, 0:1] + dp_ref[1, :, 0:1]
    inv = 1.0 / jnp.maximum(deg, 1.0)
    neigh = (np_ref[0] + np_ref[1]) * inv
    h = h_ref[...]
    z = (jnp.dot(h, ws_ref[...], preferred_element_type=jnp.float32)
         + jnp.dot(neigh, wn_ref[...], preferred_element_type=jnp.float32)
         + bl_ref[...])
    z = jnp.maximum(z, 0.0) + h
    mu = jnp.mean(z, axis=1, keepdims=True)
    var = jnp.mean((z - mu) ** 2, axis=1, keepdims=True)
    return (z - mu) / jnp.sqrt(var + 1e-5) * g_ref[...] + be_ref[...]


def _layer_body(h_ref, np_ref, dp_ref, ws_ref, wn_ref, bl_ref, g_ref, be_ref,
                o_ref):
    o_ref[...] = _sage_core(h_ref, np_ref, dp_ref, ws_ref, wn_ref, bl_ref,
                            g_ref, be_ref)


def _layer_skip_body(h_ref, np_ref, dp_ref, ws_ref, wn_ref, bl_ref, g_ref,
                     be_ref, fh_ref, wsk_ref, bsk_ref, o_ref):
    y = _sage_core(h_ref, np_ref, dp_ref, ws_ref, wn_ref, bl_ref, g_ref,
                   be_ref)
    o_ref[...] = y + jnp.dot(fh_ref[...], wsk_ref[...],
                             preferred_element_type=jnp.float32) + bsk_ref[...]


def _layer_out_body(h_ref, np_ref, dp_ref, ws_ref, wn_ref, bl_ref, g_ref,
                    be_ref, wo_ref, bo_ref, o_ref):
    y = _sage_core(h_ref, np_ref, dp_ref, ws_ref, wn_ref, bl_ref, g_ref,
                   be_ref)
    o_ref[...] = jnp.dot(y, wo_ref[...],
                         preferred_element_type=jnp.float32) + bo_ref[...]


def _base_specs():
    return [pl.BlockSpec((_BN, H), lambda i: (i, 0)),
            pl.BlockSpec((2, _BN, H), lambda i: (0, i, 0)),
            pl.BlockSpec((2, _BN, 16), lambda i: (0, i, 0)),
            pl.BlockSpec((H, H), lambda i: (0, 0)),
            pl.BlockSpec((H, H), lambda i: (0, 0)),
            pl.BlockSpec((1, H), lambda i: (0, 0)),
            pl.BlockSpec((1, H), lambda i: (0, 0)),
            pl.BlockSpec((1, H), lambda i: (0, 0))]


def _layer(h, part, degp, ws, wn, bl, g, be):
    return pl.pallas_call(
        _layer_body,
        grid=(N // _BN,),
        in_specs=_base_specs(),
        out_specs=pl.BlockSpec((_BN, H), lambda i: (i, 0)),
        out_shape=jax.ShapeDtypeStruct((N, H), jnp.float32),
    )(h, part, degp, ws, wn, bl.reshape(1, H), g.reshape(1, H),
      be.reshape(1, H))


def _layer_skip(h, part, degp, ws, wn, bl, g, be, fh, wsk, bsk):
    specs = _base_specs() + [pl.BlockSpec((_BN, H), lambda i: (i, 0)),
                             pl.BlockSpec((H, H), lambda i: (0, 0)),
                             pl.BlockSpec((1, H), lambda i: (0, 0))]
    return pl.pallas_call(
        _layer_skip_body,
        grid=(N // _BN,),
        in_specs=specs,
        out_specs=pl.BlockSpec((_BN, H), lambda i: (i, 0)),
        out_shape=jax.ShapeDtypeStruct((N, H), jnp.float32),
    )(h, part, degp, ws, wn, bl.reshape(1, H), g.reshape(1, H),
      be.reshape(1, H), fh, wsk, bsk.reshape(1, H))


def _layer_out(h, part, degp, ws, wn, bl, g, be, wo, bo):
    specs = _base_specs() + [pl.BlockSpec((H, D_OUT), lambda i: (0, 0)),
                             pl.BlockSpec((1, D_OUT), lambda i: (0, 0))]
    return pl.pallas_call(
        _layer_out_body,
        grid=(N // _BN,),
        in_specs=specs,
        out_specs=pl.BlockSpec((_BN, D_OUT), lambda i: (i, 0)),
        out_shape=jax.ShapeDtypeStruct((N, D_OUT), jnp.float32),
    )(h, part, degp, ws, wn, bl.reshape(1, H), g.reshape(1, H),
      be.reshape(1, H), wo, bo.reshape(1, D_OUT))


def kernel(x, edge_index, W_enc, b_enc, W_self_0, W_neigh_0, b_l_0, g_0, be_0,
           W_self_1, W_neigh_1, b_l_1, g_1, be_1, W_self_2, W_neigh_2, b_l_2,
           g_2, be_2, W_skip, b_skip, W_out, b_out):
    pad_src = jnp.zeros((E_PAD,), jnp.int32)
    pad_dst = N + jnp.arange(E_PAD, dtype=jnp.int32) % NTRASH
    src = jnp.concatenate([edge_index[0], pad_src]).reshape(NW, NCHUNK, CH)
    dst = jnp.concatenate([edge_index[1], pad_dst]).reshape(NW, NCHUNK, CH)
    z64 = jnp.zeros((N, H), jnp.float32)
    z16 = jnp.zeros((N, 16), jnp.float32)
    ones = jnp.ones((CH, 16), jnp.float32)

    h0 = _encoder(x, W_enc, b_enc)
    part0, degp = _agg_deg(h0, src, dst, z64, ones, z16)
    h1 = _layer(h0, part0, degp, W_self_0, W_neigh_0, b_l_0, g_0, be_0)
    (part1,) = _agg(h1, src, dst, z64)
    h2 = _layer_skip(h1, part1, degp, W_self_1, W_neigh_1, b_l_1, g_1, be_1,
                     h0, W_skip, b_skip)
    (part2,) = _agg(h2, src, dst, z64)
    out = _layer_out(h2, part2, degp, W_self_2, W_neigh_2, b_l_2, g_2, be_2,
                     W_out, b_out)
    return out


# speedup vs baseline: 2.3571x; 2.3563x over previous
import jax
import jax.numpy as jnp
from jax import lax
from jax.experimental import pallas as pl
from jax.experimental.pallas import tpu as pltpu
from jax.experimental.pallas import tpu_sc as plsc

N = 10000
E = 320000
D_IN = 128
H = 64
D_OUT = 128

NC = 2
NS = 16
NW = NC * NS
CH = 128
NCHUNK = 80
EPW = NCHUNK * CH
E_PAD = NW * EPW - E
NTRASH = 512
NACC = N + NTRASH
NPS = 624
NTAIL = N - NS * NPS

_mesh = plsc.VectorSubcoreMesh(core_axis_name="c", subcore_axis_name="s")


def _build_agg(with_deg):
    GK = 2 if with_deg else 4
    NG = NCHUNK // GK
    out_type = [jax.ShapeDtypeStruct((NC, N, H), jnp.float32)]
    scratch = [
        pltpu.VMEM((NCHUNK, CH), jnp.int32),
        pltpu.VMEM((NCHUNK, CH), jnp.int32),
        pltpu.VMEM((2, GK, CH, H), jnp.float32),
        pltpu.VMEM_SHARED((NACC, H), jnp.float32),
        pltpu.SemaphoreType.DMA,
        pltpu.SemaphoreType.DMA,
        pltpu.SemaphoreType.DMA,
        pltpu.SemaphoreType.DMA,
    ]
    if with_deg:
        out_type.append(jax.ShapeDtypeStruct((NC, N, 16), jnp.float32))
        scratch += [
            pltpu.VMEM((CH, 16), jnp.float32),
            pltpu.VMEM_SHARED((NACC, 16), jnp.float32),
        ]

    def body(*refs):
        if with_deg:
            (h_hbm, src_hbm, dst_hbm, z64_hbm, ones_hbm, z16_hbm,
             part_hbm, degp_hbm,
             src_v, dst_v, rows_v, acc_sh, sem0, sem1, ssem0, ssem1,
             ones_v, dacc_sh) = refs
        else:
            (h_hbm, src_hbm, dst_hbm, z64_hbm,
             part_hbm,
             src_v, dst_v, rows_v, acc_sh, sem0, sem1, ssem0, ssem1) = refs

        c = lax.axis_index("c")
        s = lax.axis_index("s")
        wid = c * NS + s
        r0 = pl.multiple_of(s * NPS, 8)

        pltpu.sync_copy(z64_hbm.at[pl.ds(r0, NPS)], acc_sh.at[pl.ds(r0, NPS)])
        if with_deg:
            pltpu.sync_copy(z16_hbm.at[pl.ds(r0, NPS)],
                            dacc_sh.at[pl.ds(r0, NPS)])
            pltpu.sync_copy(ones_hbm, ones_v)

        @pl.when(s == NS - 1)
        def _():
            t0 = NS * NPS
            pltpu.sync_copy(z64_hbm.at[pl.ds(t0, NTAIL)],
                            acc_sh.at[pl.ds(t0, NTAIL)])
            if with_deg:
                pltpu.sync_copy(z16_hbm.at[pl.ds(t0, NTAIL)],
                                dacc_sh.at[pl.ds(t0, NTAIL)])
        pltpu.sync_copy(src_hbm.at[wid], src_v)
        pltpu.sync_copy(dst_hbm.at[wid], dst_v)
        plsc.subcore_barrier()

        def fire_g(g, buf, sem):
            for b in range(GK):
                pltpu.async_copy(
                    h_hbm.at[src_v.at[g * GK + b]], rows_v.at[buf, b], sem)

        def wait_g(g, buf, sem):
            for b in range(GK):
                pltpu.make_async_copy(
                    h_hbm.at[src_v.at[g * GK + b]], rows_v.at[buf, b], sem
                ).wait()

        def fire_s(g, buf, sem):
            for b in range(GK):
                t = g * GK + b
                pltpu.async_copy(rows_v.at[buf, b], acc_sh.at[dst_v.at[t]],
                                 sem, add=True)
                if with_deg:
                    pltpu.async_copy(ones_v, dacc_sh.at[dst_v.at[t]], sem,
                                     add=True)

        def wait_s(g, buf, sem):
            for b in range(GK):
                t = g * GK + b
                pltpu.make_async_copy(rows_v.at[buf, b],
                                      acc_sh.at[dst_v.at[t]], sem).wait()
                if with_deg:
                    pltpu.make_async_copy(ones_v, dacc_sh.at[dst_v.at[t]],
                                          sem).wait()

        fire_g(0, 0, sem0)

        @pl.loop(0, NG - 1, step=2)
        def _(i):
            @pl.when(i > 0)
            def _():
                wait_s(i - 1, 1, ssem1)
            fire_g(i + 1, 1, sem1)
            wait_g(i, 0, sem0)
            fire_s(i, 0, ssem0)
            wait_s(i, 0, ssem0)

            @pl.when(i + 2 < NG)
            def _():
                fire_g(i + 2, 0, sem0)
            wait_g(i + 1, 1, sem1)
            fire_s(i + 1, 1, ssem1)

        wait_s(NG - 1, 1, ssem1)
        plsc.subcore_barrier()

        pltpu.sync_copy(acc_sh.at[pl.ds(r0, NPS)],
                        part_hbm.at[c, pl.ds(r0, NPS)])
        if with_deg:
            pltpu.sync_copy(dacc_sh.at[pl.ds(r0, NPS)],
                            degp_hbm.at[c, pl.ds(r0, NPS)])

        @pl.when(s == NS - 1)
        def _():
            t0 = NS * NPS
            pltpu.sync_copy(acc_sh.at[pl.ds(t0, NTAIL)],
                            part_hbm.at[c, pl.ds(t0, NTAIL)])
            if with_deg:
                pltpu.sync_copy(dacc_sh.at[pl.ds(t0, NTAIL)],
                                degp_hbm.at[c, pl.ds(t0, NTAIL)])

    return pl.kernel(body, out_type=out_type, mesh=_mesh,
                     scratch_types=scratch,
                     compiler_params=pltpu.CompilerParams(
                         use_tc_tiling_on_sc=False))


_agg_deg = _build_agg(True)
_agg = _build_agg(False)

_BN = 2000


def _enc_body(x_ref, w_ref, b_ref, o_ref):
    o_ref[...] = jnp.dot(x_ref[...], w_ref[...],
                         preferred_element_type=jnp.float32) + b_ref[...]


def _encoder(x, w, b):
    return pl.pallas_call(
        _enc_body,
        grid=(N // _BN,),
        in_specs=[pl.BlockSpec((_BN, D_IN), lambda i: (i, 0)),
                  pl.BlockSpec((D_IN, H), lambda i: (0, 0)),
                  pl.BlockSpec((1, H), lambda i: (0, 0))],
        out_specs=pl.BlockSpec((_BN, H), lambda i: (i, 0)),
        out_shape=jax.ShapeDtypeStruct((N, H), jnp.float32),
    )(x, w, b.reshape(1, H))


def _sage_core(h_ref, np_ref, dp_ref, ws_ref, wn_ref, bl_ref, g_ref, be_ref):
    deg = dp_ref[0, :, 0:1] + dp_ref[1, :, 0:1]
    inv = 1.0 / jnp.maximum(deg, 1.0)
    neigh = (np_ref[0] + np_ref[1]) * inv
    h = h_ref[...]
    z = (jnp.dot(h, ws_ref[...], preferred_element_type=jnp.float32)
         + jnp.dot(neigh, wn_ref[...], preferred_element_type=jnp.float32)
         + bl_ref[...])
    z = jnp.maximum(z, 0.0) + h
    mu = jnp.mean(z, axis=1, keepdims=True)
    var = jnp.mean((z - mu) ** 2, axis=1, keepdims=True)
    return (z - mu) / jnp.sqrt(var + 1e-5) * g_ref[...] + be_ref[...]


def _layer_body(h_ref, np_ref, dp_ref, ws_ref, wn_ref, bl_ref, g_ref, be_ref,
                o_ref):
    o_ref[...] = _sage_core(h_ref, np_ref, dp_ref, ws_ref, wn_ref, bl_ref,
                            g_ref, be_ref)


def _layer_skip_body(h_ref, np_ref, dp_ref, ws_ref, wn_ref, bl_ref, g_ref,
                     be_ref, fh_ref, wsk_ref, bsk_ref, o_ref):
    y = _sage_core(h_ref, np_ref, dp_ref, ws_ref, wn_ref, bl_ref, g_ref,
                   be_ref)
    o_ref[...] = y + jnp.dot(fh_ref[...], wsk_ref[...],
                             preferred_element_type=jnp.float32) + bsk_ref[...]


def _layer_out_body(h_ref, np_ref, dp_ref, ws_ref, wn_ref, bl_ref, g_ref,
                    be_ref, wo_ref, bo_ref, o_ref):
    y = _sage_core(h_ref, np_ref, dp_ref, ws_ref, wn_ref, bl_ref, g_ref,
                   be_ref)
    o_ref[...] = jnp.dot(y, wo_ref[...],
                         preferred_element_type=jnp.float32) + bo_ref[...]


def _base_specs():
    return [pl.BlockSpec((_BN, H), lambda i: (i, 0)),
            pl.BlockSpec((2, _BN, H), lambda i: (0, i, 0)),
            pl.BlockSpec((2, _BN, 16), lambda i: (0, i, 0)),
            pl.BlockSpec((H, H), lambda i: (0, 0)),
            pl.BlockSpec((H, H), lambda i: (0, 0)),
            pl.BlockSpec((1, H), lambda i: (0, 0)),
            pl.BlockSpec((1, H), lambda i: (0, 0)),
            pl.BlockSpec((1, H), lambda i: (0, 0))]


def _layer(h, part, degp, ws, wn, bl, g, be):
    return pl.pallas_call(
        _layer_body,
        grid=(N // _BN,),
        in_specs=_base_specs(),
        out_specs=pl.BlockSpec((_BN, H), lambda i: (i, 0)),
        out_shape=jax.ShapeDtypeStruct((N, H), jnp.float32),
    )(h, part, degp, ws, wn, bl.reshape(1, H), g.reshape(1, H),
      be.reshape(1, H))


def _layer_skip(h, part, degp, ws, wn, bl, g, be, fh, wsk, bsk):
    specs = _base_specs() + [pl.BlockSpec((_BN, H), lambda i: (i, 0)),
                             pl.BlockSpec((H, H), lambda i: (0, 0)),
                             pl.BlockSpec((1, H), lambda i: (0, 0))]
    return pl.pallas_call(
        _layer_skip_body,
        grid=(N // _BN,),
        in_specs=specs,
        out_specs=pl.BlockSpec((_BN, H), lambda i: (i, 0)),
        out_shape=jax.ShapeDtypeStruct((N, H), jnp.float32),
    )(h, part, degp, ws, wn, bl.reshape(1, H), g.reshape(1, H),
      be.reshape(1, H), fh, wsk, bsk.reshape(1, H))


def _layer_out(h, part, degp, ws, wn, bl, g, be, wo, bo):
    specs = _base_specs() + [pl.BlockSpec((H, D_OUT), lambda i: (0, 0)),
                             pl.BlockSpec((1, D_OUT), lambda i: (0, 0))]
    return pl.pallas_call(
        _layer_out_body,
        grid=(N // _BN,),
        in_specs=specs,
        out_specs=pl.BlockSpec((_BN, D_OUT), lambda i: (i, 0)),
        out_shape=jax.ShapeDtypeStruct((N, D_OUT), jnp.float32),
    )(h, part, degp, ws, wn, bl.reshape(1, H), g.reshape(1, H),
      be.reshape(1, H), wo, bo.reshape(1, D_OUT))


def kernel(x, edge_index, W_enc, b_enc, W_self_0, W_neigh_0, b_l_0, g_0, be_0,
           W_self_1, W_neigh_1, b_l_1, g_1, be_1, W_self_2, W_neigh_2, b_l_2,
           g_2, be_2, W_skip, b_skip, W_out, b_out):
    pad_src = jnp.arange(E_PAD, dtype=jnp.int32) % N
    pad_dst = N + jnp.arange(E_PAD, dtype=jnp.int32) % NTRASH
    src = jnp.concatenate([edge_index[0], pad_src]).reshape(NW, NCHUNK, CH)
    dst = jnp.concatenate([edge_index[1], pad_dst]).reshape(NW, NCHUNK, CH)
    z64 = jnp.zeros((N, H), jnp.float32)
    z16 = jnp.zeros((N, 16), jnp.float32)
    ones = jnp.ones((CH, 16), jnp.float32)

    h0 = _encoder(x, W_enc, b_enc)
    part0, degp = _agg_deg(h0, src, dst, z64, ones, z16)
    h1 = _layer(h0, part0, degp, W_self_0, W_neigh_0, b_l_0, g_0, be_0)
    (part1,) = _agg(h1, src, dst, z64)
    h2 = _layer_skip(h1, part1, degp, W_self_1, W_neigh_1, b_l_1, g_1, be_1,
                     h0, W_skip, b_skip)
    (part2,) = _agg(h2, src, dst, z64)
    out = _layer_out(h2, part2, degp, W_self_2, W_neigh_2, b_l_2, g_2, be_2,
                     W_out, b_out)
    return out


# speedup vs baseline: 2.4844x; 1.0540x over previous
import jax
import jax.numpy as jnp
from jax import lax
from jax.experimental import pallas as pl
from jax.experimental.pallas import tpu as pltpu
from jax.experimental.pallas import tpu_sc as plsc

N = 10000
E = 320000
D_IN = 128
H = 64
D_OUT = 128

NC = 2
NS = 16
NW = NC * NS
CH = 128
NCHUNK = 80
EPW = NCHUNK * CH
E_PAD = NW * EPW - E
NTRASH = 512
NACC = N + NTRASH
NPS = 624
NTAIL = N - NS * NPS

_mesh = plsc.VectorSubcoreMesh(core_axis_name="c", subcore_axis_name="s")


def _build_agg(with_deg):
    GK = 2 if with_deg else 4
    NG = NCHUNK // GK
    out_type = [jax.ShapeDtypeStruct((NC, N, H), jnp.float32)]
    scratch = [
        pltpu.VMEM((NCHUNK, CH), jnp.int32),
        pltpu.VMEM((NCHUNK, CH), jnp.int32),
        pltpu.VMEM((2, GK, CH, H), jnp.float32),
        pltpu.VMEM_SHARED((NACC, H), jnp.float32),
        pltpu.SemaphoreType.DMA,
        pltpu.SemaphoreType.DMA,
        pltpu.SemaphoreType.DMA,
        pltpu.SemaphoreType.DMA,
    ]
    if with_deg:
        out_type.append(jax.ShapeDtypeStruct((NC, N, 16), jnp.float32))
        scratch += [
            pltpu.VMEM((CH, 16), jnp.float32),
            pltpu.VMEM_SHARED((NACC, 16), jnp.float32),
        ]

    def body(*refs):
        if with_deg:
            (h_hbm, src_hbm, dst_hbm, z64_hbm, ones_hbm, z16_hbm,
             part_hbm, degp_hbm,
             src_v, dst_v, rows_v, acc_sh, sem0, sem1, ssem0, ssem1,
             ones_v, dacc_sh) = refs
        else:
            (h_hbm, src_hbm, dst_hbm, z64_hbm,
             part_hbm,
             src_v, dst_v, rows_v, acc_sh, sem0, sem1, ssem0, ssem1) = refs

        c = lax.axis_index("c")
        s = lax.axis_index("s")
        wid = c * NS + s
        r0 = pl.multiple_of(s * NPS, 8)

        pltpu.sync_copy(z64_hbm.at[pl.ds(r0, NPS)], acc_sh.at[pl.ds(r0, NPS)])
        if with_deg:
            pltpu.sync_copy(z16_hbm.at[pl.ds(r0, NPS)],
                            dacc_sh.at[pl.ds(r0, NPS)])
            pltpu.sync_copy(ones_hbm, ones_v)

        @pl.when(s == NS - 1)
        def _():
            t0 = NS * NPS
            pltpu.sync_copy(z64_hbm.at[pl.ds(t0, NTAIL)],
                            acc_sh.at[pl.ds(t0, NTAIL)])
            if with_deg:
                pltpu.sync_copy(z16_hbm.at[pl.ds(t0, NTAIL)],
                                dacc_sh.at[pl.ds(t0, NTAIL)])
        pltpu.sync_copy(src_hbm.at[wid], src_v)
        pltpu.sync_copy(dst_hbm.at[wid], dst_v)
        plsc.subcore_barrier()

        def fire_g(g, buf, sem):
            for b in range(GK):
                pltpu.async_copy(
                    h_hbm.at[src_v.at[g * GK + b]], rows_v.at[buf, b], sem)

        def wait_g(g, buf, sem):
            for b in range(GK):
                pltpu.make_async_copy(
                    h_hbm.at[src_v.at[g * GK + b]], rows_v.at[buf, b], sem
                ).wait()

        def fire_s(g, buf, sem):
            for b in range(GK):
                t = g * GK + b
                pltpu.async_copy(rows_v.at[buf, b], acc_sh.at[dst_v.at[t]],
                                 sem, add=True)
                if with_deg:
                    pltpu.async_copy(ones_v, dacc_sh.at[dst_v.at[t]], sem,
                                     add=True)

        def wait_s(g, buf, sem):
            for b in range(GK):
                t = g * GK + b
                pltpu.make_async_copy(rows_v.at[buf, b],
                                      acc_sh.at[dst_v.at[t]], sem).wait()
                if with_deg:
                    pltpu.make_async_copy(ones_v, dacc_sh.at[dst_v.at[t]],
                                          sem).wait()

        fire_g(0, 0, sem0)

        @pl.loop(0, NG - 1, step=2)
        def _(i):
            @pl.when(i > 0)
            def _():
                wait_s(i - 1, 1, ssem1)
            fire_g(i + 1, 1, sem1)
            wait_g(i, 0, sem0)
            fire_s(i, 0, ssem0)
            wait_s(i, 0, ssem0)

            @pl.when(i + 2 < NG)
            def _():
                fire_g(i + 2, 0, sem0)
            wait_g(i + 1, 1, sem1)
            fire_s(i + 1, 1, ssem1)

        wait_s(NG - 1, 1, ssem1)
        plsc.subcore_barrier()

        pltpu.sync_copy(acc_sh.at[pl.ds(r0, NPS)],
                        part_hbm.at[c, pl.ds(r0, NPS)])
        if with_deg:
            pltpu.sync_copy(dacc_sh.at[pl.ds(r0, NPS)],
                            degp_hbm.at[c, pl.ds(r0, NPS)])

        @pl.when(s == NS - 1)
        def _():
            t0 = NS * NPS
            pltpu.sync_copy(acc_sh.at[pl.ds(t0, NTAIL)],
                            part_hbm.at[c, pl.ds(t0, NTAIL)])
            if with_deg:
                pltpu.sync_copy(dacc_sh.at[pl.ds(t0, NTAIL)],
                                degp_hbm.at[c, pl.ds(t0, NTAIL)])

    return pl.kernel(body, out_type=out_type, mesh=_mesh,
                     scratch_types=scratch,
                     compiler_params=pltpu.CompilerParams(
                         use_tc_tiling_on_sc=False))


_agg_deg = _build_agg(True)
_agg = _build_agg(False)

_BP = 1000
_GRID = N // 2 // _BP


def _enc_body(x_ref, w_ref, b_ref, o_ref):
    o_ref[...] = jnp.dot(x_ref[...], w_ref[...],
                         preferred_element_type=jnp.float32) + b_ref[...]


def _encoder(x2, w2, b2):
    return pl.pallas_call(
        _enc_body,
        grid=(_GRID,),
        in_specs=[pl.BlockSpec((_BP, 2 * D_IN), lambda i: (i, 0)),
                  pl.BlockSpec((2 * D_IN, 128), lambda i: (0, 0)),
                  pl.BlockSpec((1, 128), lambda i: (0, 0))],
        out_specs=pl.BlockSpec((_BP, 128), lambda i: (i, 0)),
        out_shape=jax.ShapeDtypeStruct((N // 2, 128), jnp.float32),
    )(x2, w2, b2)


def _sage_core(h_ref, np_ref, db_ref, m_ref, ws_ref, wn_ref, bl_ref, g_ref,
               be_ref):
    inv = 1.0 / jnp.maximum(db_ref[0] + db_ref[1], 1.0)
    neigh = (np_ref[0] + np_ref[1]) * inv
    h = h_ref[...]
    z = (jnp.dot(h, ws_ref[...], preferred_element_type=jnp.float32)
         + jnp.dot(neigh, wn_ref[...], preferred_element_type=jnp.float32)
         + bl_ref[...])
    z = jnp.maximum(z, 0.0) + h
    mu = jnp.dot(z, m_ref[...], preferred_element_type=jnp.float32)
    zc = z - mu
    var = jnp.dot(zc * zc, m_ref[...], preferred_element_type=jnp.float32)
    return zc * jax.lax.rsqrt(var + 1e-5) * g_ref[...] + be_ref[...]


def _layer_body(h_ref, np_ref, db_ref, m_ref, ws_ref, wn_ref, bl_ref, g_ref,
                be_ref, o_ref):
    o_ref[...] = _sage_core(h_ref, np_ref, db_ref, m_ref, ws_ref, wn_ref,
                            bl_ref, g_ref, be_ref)


def _layer_skip_body(h_ref, np_ref, db_ref, m_ref, ws_ref, wn_ref, bl_ref,
                     g_ref, be_ref, fh_ref, wsk_ref, bsk_ref, o_ref):
    y = _sage_core(h_ref, np_ref, db_ref, m_ref, ws_ref, wn_ref, bl_ref,
                   g_ref, be_ref)
    o_ref[...] = y + jnp.dot(fh_ref[...], wsk_ref[...],
                             preferred_element_type=jnp.float32) + bsk_ref[...]


def _layer_out_body(h_ref, np_ref, db_ref, m_ref, ws_ref, wn_ref, bl_ref,
                    g_ref, be_ref, wo_ref, bo_ref, o_ref):
    y = _sage_core(h_ref, np_ref, db_ref, m_ref, ws_ref, wn_ref, bl_ref,
                   g_ref, be_ref)
    o_ref[...] = jnp.dot(y, wo_ref[...],
                         preferred_element_type=jnp.float32) + bo_ref[...]


def _base_specs():
    return [pl.BlockSpec((_BP, 128), lambda i: (i, 0)),
            pl.BlockSpec((2, _BP, 128), lambda i: (0, i, 0)),
            pl.BlockSpec((2, _BP, 128), lambda i: (0, i, 0)),
            pl.BlockSpec((128, 128), lambda i: (0, 0)),
            pl.BlockSpec((128, 128), lambda i: (0, 0)),
            pl.BlockSpec((128, 128), lambda i: (0, 0)),
            pl.BlockSpec((1, 128), lambda i: (0, 0)),
            pl.BlockSpec((1, 128), lambda i: (0, 0)),
            pl.BlockSpec((1, 128), lambda i: (0, 0))]


def _layer(h, part2, degb, m, w2s, w2n, bl2, g2, be2):
    return pl.pallas_call(
        _layer_body,
        grid=(_GRID,),
        in_specs=_base_specs(),
        out_specs=pl.BlockSpec((_BP, 128), lambda i: (i, 0)),
        out_shape=jax.ShapeDtypeStruct((N // 2, 128), jnp.float32),
    )(h, part2, degb, m, w2s, w2n, bl2, g2, be2)


def _layer_skip(h, part2, degb, m, w2s, w2n, bl2, g2, be2, fh, w2sk, bsk2):
    specs = _base_specs() + [pl.BlockSpec((_BP, 128), lambda i: (i, 0)),
                             pl.BlockSpec((128, 128), lambda i: (0, 0)),
                             pl.BlockSpec((1, 128), lambda i: (0, 0))]
    return pl.pallas_call(
        _layer_skip_body,
        grid=(_GRID,),
        in_specs=specs,
        out_specs=pl.BlockSpec((_BP, 128), lambda i: (i, 0)),
        out_shape=jax.ShapeDtypeStruct((N // 2, 128), jnp.float32),
    )(h, part2, degb, m, w2s, w2n, bl2, g2, be2, fh, w2sk, bsk2)


def _layer_out(h, part2, degb, m, w2s, w2n, bl2, g2, be2, w2o, bo2):
    specs = _base_specs() + [pl.BlockSpec((128, 2 * D_OUT), lambda i: (0, 0)),
                             pl.BlockSpec((1, 2 * D_OUT), lambda i: (0, 0))]
    return pl.pallas_call(
        _layer_out_body,
        grid=(_GRID,),
        in_specs=specs,
        out_specs=pl.BlockSpec((_BP, 2 * D_OUT), lambda i: (i, 0)),
        out_shape=jax.ShapeDtypeStruct((N // 2, 2 * D_OUT), jnp.float32),
    )(h, part2, degb, m, w2s, w2n, bl2, g2, be2, w2o, bo2)


def _bd(w):
    m, n = w.shape
    z = jnp.zeros((2 * m, 2 * n), w.dtype)
    return z.at[:m, :n].set(w).at[m:, n:].set(w)


def _t2(v):
    return jnp.tile(v, 2).reshape(1, -1)


def kernel(x, edge_index, W_enc, b_enc, W_self_0, W_neigh_0, b_l_0, g_0, be_0,
           W_self_1, W_neigh_1, b_l_1, g_1, be_1, W_self_2, W_neigh_2, b_l_2,
           g_2, be_2, W_skip, b_skip, W_out, b_out):
    pad_src = jnp.arange(E_PAD, dtype=jnp.int32) % N
    pad_dst = N + jnp.arange(E_PAD, dtype=jnp.int32) % NTRASH
    src = jnp.concatenate([edge_index[0], pad_src]).reshape(NW, NCHUNK, CH)
    dst = jnp.concatenate([edge_index[1], pad_dst]).reshape(NW, NCHUNK, CH)
    z64 = jnp.zeros((N, H), jnp.float32)
    z16 = jnp.zeros((N, 16), jnp.float32)
    ones = jnp.ones((CH, 16), jnp.float32)
    m = _bd(jnp.full((H, H), 1.0 / H, jnp.float32))

    x2 = x.reshape(N // 2, 2 * D_IN)
    h0 = _encoder(x2, _bd(W_enc), _t2(b_enc))
    part0, degp = _agg_deg(h0.reshape(N, H), src, dst, z64, ones, z16)
    part0 = part0.reshape(NC, N // 2, 128)
    degb = jnp.broadcast_to(degp[:, :, 0:1], (NC, N, H))
    degb = degb.reshape(NC, N // 2, 128)
    h1 = _layer(h0, part0, degb, m, _bd(W_self_0), _bd(W_neigh_0),
                _t2(b_l_0), _t2(g_0), _t2(be_0))
    (part1,) = _agg(h1.reshape(N, H), src, dst, z64)
    h2 = _layer_skip(h1, part1.reshape(NC, N // 2, 128), degb, m,
                     _bd(W_self_1), _bd(W_neigh_1), _t2(b_l_1), _t2(g_1),
                     _t2(be_1), h0, _bd(W_skip), _t2(b_skip))
    (part2,) = _agg(h2.reshape(N, H), src, dst, z64)
    out = _layer_out(h2, part2.reshape(NC, N // 2, 128), degb, m,
                     _bd(W_self_2), _bd(W_neigh_2), _t2(b_l_2), _t2(g_2),
                     _t2(be_2), _bd(W_out), _t2(b_out))
    return out.reshape(N, D_OUT)


# speedup vs baseline: 2.5963x; 1.0450x over previous
import jax
import jax.numpy as jnp
from jax import lax
from jax.experimental import pallas as pl
from jax.experimental.pallas import tpu as pltpu
from jax.experimental.pallas import tpu_sc as plsc

N = 10000
E = 320000
D_IN = 128
H = 64
D_OUT = 128

NC = 2
NS = 16
NW = NC * NS
CH = 128
NCHUNK = 80
EPW = NCHUNK * CH
E_PAD = NW * EPW - E
NTRASH = 512
NACC = N + NTRASH
NPS = 624
NTAIL = N - NS * NPS

_mesh = plsc.VectorSubcoreMesh(core_axis_name="c", subcore_axis_name="s")


def _build_agg(with_deg):
    GK = 2 if with_deg else 4
    NG = NCHUNK // GK
    out_type = [jax.ShapeDtypeStruct((NC, N, H), jnp.float32)]
    scratch = [
        pltpu.VMEM((NCHUNK, CH), jnp.int32),
        pltpu.VMEM((NCHUNK, CH), jnp.int32),
        pltpu.VMEM((2, GK, CH, H), jnp.float32),
        pltpu.VMEM_SHARED((NACC, H), jnp.float32),
        pltpu.SemaphoreType.DMA,
        pltpu.SemaphoreType.DMA,
        pltpu.SemaphoreType.DMA,
        pltpu.SemaphoreType.DMA,
    ]
    if with_deg:
        out_type.append(jax.ShapeDtypeStruct((NC, N, H), jnp.float32))
        scratch += [
            pltpu.VMEM((CH, 16), jnp.float32),
            pltpu.VMEM_SHARED((NACC, 16), jnp.float32),
        ]

    def body(*refs):
        if with_deg:
            (h_hbm, src_hbm, dst_hbm, z64_hbm, ones_hbm, z16_hbm,
             part_hbm, degp_hbm,
             src_v, dst_v, rows_v, acc_sh, sem0, sem1, ssem0, ssem1,
             ones_v, dacc_sh) = refs
        else:
            (h_hbm, src_hbm, dst_hbm, z64_hbm,
             part_hbm,
             src_v, dst_v, rows_v, acc_sh, sem0, sem1, ssem0, ssem1) = refs

        c = lax.axis_index("c")
        s = lax.axis_index("s")
        wid = c * NS + s
        r0 = pl.multiple_of(s * NPS, 8)

        pltpu.sync_copy(z64_hbm.at[pl.ds(r0, NPS)], acc_sh.at[pl.ds(r0, NPS)])
        if with_deg:
            pltpu.sync_copy(z16_hbm.at[pl.ds(r0, NPS)],
                            dacc_sh.at[pl.ds(r0, NPS)])
            pltpu.sync_copy(ones_hbm, ones_v)

        @pl.when(s == NS - 1)
        def _():
            t0 = NS * NPS
            pltpu.sync_copy(z64_hbm.at[pl.ds(t0, NTAIL)],
                            acc_sh.at[pl.ds(t0, NTAIL)])
            if with_deg:
                pltpu.sync_copy(z16_hbm.at[pl.ds(t0, NTAIL)],
                                dacc_sh.at[pl.ds(t0, NTAIL)])
        pltpu.sync_copy(src_hbm.at[wid], src_v)
        pltpu.sync_copy(dst_hbm.at[wid], dst_v)
        plsc.subcore_barrier()

        def fire_g(g, buf, sem):
            for b in range(GK):
                pltpu.async_copy(
                    h_hbm.at[src_v.at[g * GK + b]], rows_v.at[buf, b], sem)

        def wait_g(g, buf, sem):
            for b in range(GK):
                pltpu.make_async_copy(
                    h_hbm.at[src_v.at[g * GK + b]], rows_v.at[buf, b], sem
                ).wait()

        def fire_s(g, buf, sem):
            for b in range(GK):
                t = g * GK + b
                pltpu.async_copy(rows_v.at[buf, b], acc_sh.at[dst_v.at[t]],
                                 sem, add=True)
                if with_deg:
                    pltpu.async_copy(ones_v, dacc_sh.at[dst_v.at[t]], sem,
                                     add=True)

        def wait_s(g, buf, sem):
            for b in range(GK):
                t = g * GK + b
                pltpu.make_async_copy(rows_v.at[buf, b],
                                      acc_sh.at[dst_v.at[t]], sem).wait()
                if with_deg:
                    pltpu.make_async_copy(ones_v, dacc_sh.at[dst_v.at[t]],
                                          sem).wait()

        fire_g(0, 0, sem0)

        @pl.loop(0, NG - 1, step=2)
        def _(i):
            @pl.when(i > 0)
            def _():
                wait_s(i - 1, 1, ssem1)
            fire_g(i + 1, 1, sem1)
            wait_g(i, 0, sem0)
            fire_s(i, 0, ssem0)
            wait_s(i, 0, ssem0)

            @pl.when(i + 2 < NG)
            def _():
                fire_g(i + 2, 0, sem0)
            wait_g(i + 1, 1, sem1)
            fire_s(i + 1, 1, ssem1)

        wait_s(NG - 1, 1, ssem1)
        plsc.subcore_barrier()

        pltpu.sync_copy(acc_sh.at[pl.ds(r0, NPS)],
                        part_hbm.at[c, pl.ds(r0, NPS)])
        if with_deg:
            for j in range(H // 16):
                pltpu.sync_copy(
                    dacc_sh.at[pl.ds(r0, NPS)],
                    degp_hbm.at[c, pl.ds(r0, NPS), pl.ds(16 * j, 16)])

        @pl.when(s == NS - 1)
        def _():
            t0 = NS * NPS
            pltpu.sync_copy(acc_sh.at[pl.ds(t0, NTAIL)],
                            part_hbm.at[c, pl.ds(t0, NTAIL)])
            if with_deg:
                for j in range(H // 16):
                    pltpu.sync_copy(
                        dacc_sh.at[pl.ds(t0, NTAIL)],
                        degp_hbm.at[c, pl.ds(t0, NTAIL), pl.ds(16 * j, 16)])

    return pl.kernel(body, out_type=out_type, mesh=_mesh,
                     scratch_types=scratch,
                     compiler_params=pltpu.CompilerParams(
                         use_tc_tiling_on_sc=False))


_agg_deg = _build_agg(True)
_agg = _build_agg(False)

_BP = 1000
_GRID = N // 2 // _BP


def _enc_body(x_ref, w_ref, b_ref, o_ref):
    o_ref[...] = jnp.dot(x_ref[...], w_ref[...],
                         preferred_element_type=jnp.float32) + b_ref[...]


def _encoder(x2, w2, b2):
    return pl.pallas_call(
        _enc_body,
        grid=(_GRID,),
        in_specs=[pl.BlockSpec((_BP, 2 * D_IN), lambda i: (i, 0)),
                  pl.BlockSpec((2 * D_IN, 128), lambda i: (0, 0)),
                  pl.BlockSpec((1, 128), lambda i: (0, 0))],
        out_specs=pl.BlockSpec((_BP, 128), lambda i: (i, 0)),
        out_shape=jax.ShapeDtypeStruct((N // 2, 128), jnp.float32),
    )(x2, w2, b2)


def _sage_core(h_ref, np_ref, db_ref, m_ref, ws_ref, wn_ref, bl_ref, g_ref,
               be_ref):
    inv = 1.0 / jnp.maximum(db_ref[0] + db_ref[1], 1.0)
    neigh = (np_ref[0] + np_ref[1]) * inv
    h = h_ref[...]
    z = (jnp.dot(h, ws_ref[...], preferred_element_type=jnp.float32)
         + jnp.dot(neigh, wn_ref[...], preferred_element_type=jnp.float32)
         + bl_ref[...])
    z = jnp.maximum(z, 0.0) + h
    mu = jnp.dot(z, m_ref[...], preferred_element_type=jnp.float32)
    zc = z - mu
    var = jnp.dot(zc * zc, m_ref[...], preferred_element_type=jnp.float32)
    return zc * jax.lax.rsqrt(var + 1e-5) * g_ref[...] + be_ref[...]


def _layer_body(h_ref, np_ref, db_ref, m_ref, ws_ref, wn_ref, bl_ref, g_ref,
                be_ref, o_ref):
    o_ref[...] = _sage_core(h_ref, np_ref, db_ref, m_ref, ws_ref, wn_ref,
                            bl_ref, g_ref, be_ref)


def _layer_skip_body(h_ref, np_ref, db_ref, m_ref, ws_ref, wn_ref, bl_ref,
                     g_ref, be_ref, fh_ref, wsk_ref, bsk_ref, o_ref):
    y = _sage_core(h_ref, np_ref, db_ref, m_ref, ws_ref, wn_ref, bl_ref,
                   g_ref, be_ref)
    o_ref[...] = y + jnp.dot(fh_ref[...], wsk_ref[...],
                             preferred_element_type=jnp.float32) + bsk_ref[...]


def _layer_out_body(h_ref, np_ref, db_ref, m_ref, ws_ref, wn_ref, bl_ref,
                    g_ref, be_ref, wo_ref, bo_ref, o_ref):
    y = _sage_core(h_ref, np_ref, db_ref, m_ref, ws_ref, wn_ref, bl_ref,
                   g_ref, be_ref)
    o_ref[...] = jnp.dot(y, wo_ref[...],
                         preferred_element_type=jnp.float32) + bo_ref[...]


def _base_specs():
    return [pl.BlockSpec((_BP, 128), lambda i: (i, 0)),
            pl.BlockSpec((2, _BP, 128), lambda i: (0, i, 0)),
            pl.BlockSpec((2, _BP, 128), lambda i: (0, i, 0)),
            pl.BlockSpec((128, 128), lambda i: (0, 0)),
            pl.BlockSpec((128, 128), lambda i: (0, 0)),
            pl.BlockSpec((128, 128), lambda i: (0, 0)),
            pl.BlockSpec((1, 128), lambda i: (0, 0)),
            pl.BlockSpec((1, 128), lambda i: (0, 0)),
            pl.BlockSpec((1, 128), lambda i: (0, 0))]


def _layer(h, part2, degb, m, w2s, w2n, bl2, g2, be2):
    return pl.pallas_call(
        _layer_body,
        grid=(_GRID,),
        in_specs=_base_specs(),
        out_specs=pl.BlockSpec((_BP, 128), lambda i: (i, 0)),
        out_shape=jax.ShapeDtypeStruct((N // 2, 128), jnp.float32),
    )(h, part2, degb, m, w2s, w2n, bl2, g2, be2)


def _layer_skip(h, part2, degb, m, w2s, w2n, bl2, g2, be2, fh, w2sk, bsk2):
    specs = _base_specs() + [pl.BlockSpec((_BP, 128), lambda i: (i, 0)),
                             pl.BlockSpec((128, 128), lambda i: (0, 0)),
                             pl.BlockSpec((1, 128), lambda i: (0, 0))]
    return pl.pallas_call(
        _layer_skip_body,
        grid=(_GRID,),
        in_specs=specs,
        out_specs=pl.BlockSpec((_BP, 128), lambda i: (i, 0)),
        out_shape=jax.ShapeDtypeStruct((N // 2, 128), jnp.float32),
    )(h, part2, degb, m, w2s, w2n, bl2, g2, be2, fh, w2sk, bsk2)


def _layer_out(h, part2, degb, m, w2s, w2n, bl2, g2, be2, w2o, bo2):
    specs = _base_specs() + [pl.BlockSpec((128, 2 * D_OUT), lambda i: (0, 0)),
                             pl.BlockSpec((1, 2 * D_OUT), lambda i: (0, 0))]
    return pl.pallas_call(
        _layer_out_body,
        grid=(_GRID,),
        in_specs=specs,
        out_specs=pl.BlockSpec((_BP, 2 * D_OUT), lambda i: (i, 0)),
        out_shape=jax.ShapeDtypeStruct((N // 2, 2 * D_OUT), jnp.float32),
    )(h, part2, degb, m, w2s, w2n, bl2, g2, be2, w2o, bo2)


def _bd(w):
    m, n = w.shape
    z = jnp.zeros((2 * m, 2 * n), w.dtype)
    return z.at[:m, :n].set(w).at[m:, n:].set(w)


def _t2(v):
    return jnp.tile(v, 2).reshape(1, -1)


def kernel(x, edge_index, W_enc, b_enc, W_self_0, W_neigh_0, b_l_0, g_0, be_0,
           W_self_1, W_neigh_1, b_l_1, g_1, be_1, W_self_2, W_neigh_2, b_l_2,
           g_2, be_2, W_skip, b_skip, W_out, b_out):
    pad_src = jnp.arange(E_PAD, dtype=jnp.int32) % N
    pad_dst = N + jnp.arange(E_PAD, dtype=jnp.int32) % NTRASH
    src = jnp.concatenate([edge_index[0], pad_src]).reshape(NW, NCHUNK, CH)
    dst = jnp.concatenate([edge_index[1], pad_dst]).reshape(NW, NCHUNK, CH)
    z64 = jnp.zeros((N, H), jnp.float32)
    z16 = jnp.zeros((N, 16), jnp.float32)
    ones = jnp.ones((CH, 16), jnp.float32)
    m = _bd(jnp.full((H, H), 1.0 / H, jnp.float32))

    x2 = x.reshape(N // 2, 2 * D_IN)
    h0 = _encoder(x2, _bd(W_enc), _t2(b_enc))
    part0, degp = _agg_deg(h0.reshape(N, H), src, dst, z64, ones, z16)
    part0 = part0.reshape(NC, N // 2, 128)
    degb = degp.reshape(NC, N // 2, 128)
    h1 = _layer(h0, part0, degb, m, _bd(W_self_0), _bd(W_neigh_0),
                _t2(b_l_0), _t2(g_0), _t2(be_0))
    (part1,) = _agg(h1.reshape(N, H), src, dst, z64)
    h2 = _layer_skip(h1, part1.reshape(NC, N // 2, 128), degb, m,
                     _bd(W_self_1), _bd(W_neigh_1), _t2(b_l_1), _t2(g_1),
                     _t2(be_1), h0, _bd(W_skip), _t2(b_skip))
    (part2,) = _agg(h2.reshape(N, H), src, dst, z64)
    out = _layer_out(h2, part2.reshape(NC, N // 2, 128), degb, m,
                     _bd(W_self_2), _bd(W_neigh_2), _t2(b_l_2), _t2(g_2),
                     _t2(be_2), _bd(W_out), _t2(b_out))
    return out.reshape(N, D_OUT)
